# baseline (speedup 1.0000x reference)
import jax
import jax.numpy as jnp
from jax import lax
from jax.experimental import pallas as pl
from jax.experimental.pallas import tpu as pltpu
from jax.experimental.pallas import tpu_sc as plsc

B = 16
C = 6
HW = 512 * 512
NV = 100000
NC = 2
NS = 16
NW = NC * NS
CH = 3200
NVP = NW * CH
VC = 64
NCH = CH // VC
D = 48
TD = 128


def _blend_chunk(o_v, f_v, b_v, wf_v, wb_v, cb, zeros):
    def blend(r, cc):
        w1 = wf_v[pl.ds(cb + r, 16)].at[zeros].get(mode="promise_in_bounds")
        w2 = wb_v[pl.ds(cb + r, 16)].at[zeros].get(mode="promise_in_bounds")
        for j in range(3):
            s = pl.ds(r * D + j * 16, 16)
            o_v[s] = (f_v[r, pl.ds(j * 16, 16)] * w1
                      + b_v[r, pl.ds(D + j * 16, 16)] * w2)
        return cc

    lax.fori_loop(0, VC, blend, 0)


def _body(tab, idxf, idxb, mf, mb, out,
          idxf_v, idxb_v, mf_v, mb_v, wf_v, wb_v,
          f0_v, b0_v, f1_v, b1_v, o_v,
          sf0, sb0, sf1, sb1):
    cid = lax.axis_index("c")
    sid = lax.axis_index("s")
    wid = sid * NC + cid
    base = wid * CH

    pltpu.sync_copy(idxf.at[pl.ds(base, CH)], idxf_v)
    pltpu.sync_copy(idxb.at[pl.ds(base, CH)], idxb_v)
    pltpu.sync_copy(mf.at[pl.ds(base, CH)], mf_v)
    pltpu.sync_copy(mb.at[pl.ds(base, CH)], mb_v)

    def wloop(i, carry):
        s = pl.ds(i * 16, 16)
        a = lax.convert_element_type(mf_v[s], jnp.float32)
        bb = lax.convert_element_type(mb_v[s], jnp.float32)
        d = jnp.maximum(a + bb, 1.0)
        wf_v[s] = a / d
        wb_v[s] = bb / d
        return carry

    lax.fori_loop(0, CH // 16, wloop, 0)

    zeros = lax.shift_right_logical(lax.iota(jnp.int32, 16), 31)
    bufs = ((f0_v, b0_v, sf0, sb0), (f1_v, b1_v, sf1, sb1))

    def issue(k, phase):
        fv, bv, sf, sb = bufs[phase]
        cb = k * VC
        pltpu.async_copy(tab.at[idxf_v.at[pl.ds(cb, VC)]], fv, sf)
        pltpu.async_copy(tab.at[idxb_v.at[pl.ds(cb, VC)]], bv, sb)

    issue(0, 0)
    issue(1, 1)

    def step(i, carry):
        for phase in range(2):
            k = i * 2 + phase
            fv, bv, sf, sb = bufs[phase]
            cb = k * VC
            pltpu.make_async_copy(tab.at[idxf_v.at[pl.ds(cb, VC)]], fv, sf).wait()
            pltpu.make_async_copy(tab.at[idxb_v.at[pl.ds(cb, VC)]], bv, sb).wait()
            _blend_chunk(o_v, fv, bv, wf_v, wb_v, cb, zeros)
            pltpu.sync_copy(o_v, out.at[pl.ds((base + cb) * D, VC * D)])

            @pl.when(k + 2 < NCH)
            def _():
                issue(k + 2, phase)

        return carry

    lax.fori_loop(0, NCH // 2, step, 0)


def _make_sample():
    return pl.kernel(
        _body,
        mesh=plsc.VectorSubcoreMesh(core_axis_name="c", subcore_axis_name="s"),
        out_type=jax.ShapeDtypeStruct((NVP * D,), jnp.float32),
        scratch_types=[
            pltpu.VMEM((CH,), jnp.int32),
            pltpu.VMEM((CH,), jnp.int32),
            pltpu.VMEM((CH,), jnp.int32),
            pltpu.VMEM((CH,), jnp.int32),
            pltpu.VMEM((CH + 16,), jnp.float32),
            pltpu.VMEM((CH + 16,), jnp.float32),
            pltpu.VMEM((VC, TD), jnp.float32),
            pltpu.VMEM((VC, TD), jnp.float32),
            pltpu.VMEM((VC, TD), jnp.float32),
            pltpu.VMEM((VC, TD), jnp.float32),
            pltpu.VMEM((VC * D,), jnp.float32),
            pltpu.SemaphoreType.DMA,
            pltpu.SemaphoreType.DMA,
            pltpu.SemaphoreType.DMA,
            pltpu.SemaphoreType.DMA,
        ],
    )


TT = 1024


def _tc_build(img_ref, out_ref):
    for c in range(C):
        out_ref[:, pl.ds(c * B, B)] = jnp.transpose(img_ref[:, c, :], (1, 0))
    out_ref[:, pl.ds(C * B, TD - C * B)] = jnp.zeros((TT, TD - C * B), jnp.float32)


def _build_table(img):
    return pl.pallas_call(
        _tc_build,
        grid=(HW // TT,),
        in_specs=[pl.BlockSpec((B, C, TT), lambda i: (0, 0, i))],
        out_specs=pl.BlockSpec((TT, TD), lambda i: (i, 0)),
        out_shape=jax.ShapeDtypeStruct((HW, TD), jnp.float32),
    )(img)


def kernel(offset_imgs, vt_idx_front, vt_idx_back, mask_front, mask_back):
    tab = _build_table(offset_imgs.reshape(B, C, HW))
    pad = NVP - NV
    idxf = jnp.pad(vt_idx_front, (0, pad), mode="wrap")
    idxb = jnp.pad(vt_idx_back, (0, pad), mode="wrap")
    mf = jnp.pad(mask_front, (0, pad))
    mb = jnp.pad(mask_back, (0, pad))
    o = _make_sample()(tab, idxf, idxb, mf, mb)
    return jnp.transpose(o.reshape(NVP, 3, B), (2, 0, 1))[:, :NV, :]


# speedup vs baseline: 1.2393x; 1.2393x over previous
import jax
import jax.numpy as jnp
from jax import lax
from jax.experimental import pallas as pl
from jax.experimental.pallas import tpu as pltpu
from jax.experimental.pallas import tpu_sc as plsc

B = 16
C = 6
HW = 512 * 512
NV = 100000
NC = 2
NS = 16
NW = NC * NS
CH = 3200
NVP = NW * CH
VC = 64
NCH = CH // VC
D = 48
TD = 128


def _blend_chunk(o_v, f_v, b_v, wf_v, wb_v, cb, zeros):
    def blend(r, cc):
        w1 = wf_v[pl.ds(cb + r, 16)].at[zeros].get(mode="promise_in_bounds")
        w2 = wb_v[pl.ds(cb + r, 16)].at[zeros].get(mode="promise_in_bounds")
        for j in range(3):
            s = pl.ds(r * D + j * 16, 16)
            o_v[s] = (f_v[r, pl.ds(j * 16, 16)] * w1
                      + b_v[r, pl.ds(D + j * 16, 16)] * w2)
        return cc

    lax.fori_loop(0, VC, blend, 0)


def _body(tab, idxf, idxb, mf, mb, out,
          idxf_v, idxb_v, mf_v, mb_v, wf_v, wb_v,
          f0_v, b0_v, f1_v, b1_v, o_v,
          sf0, sb0, sf1, sb1):
    cid = lax.axis_index("c")
    sid = lax.axis_index("s")
    wid = sid * NC + cid
    base = wid * CH

    pltpu.sync_copy(idxf.at[pl.ds(base, CH)], idxf_v)
    pltpu.sync_copy(idxb.at[pl.ds(base, CH)], idxb_v)
    pltpu.sync_copy(mf.at[pl.ds(base, CH)], mf_v)
    pltpu.sync_copy(mb.at[pl.ds(base, CH)], mb_v)

    def wloop(i, carry):
        s = pl.ds(i * 16, 16)
        a = lax.convert_element_type(mf_v[s], jnp.float32)
        bb = lax.convert_element_type(mb_v[s], jnp.float32)
        d = jnp.maximum(a + bb, 1.0)
        wf_v[s] = a / d
        wb_v[s] = bb / d
        return carry

    lax.fori_loop(0, CH // 16, wloop, 0)

    zeros = lax.shift_right_logical(lax.iota(jnp.int32, 16), 31)
    bufs = ((f0_v, b0_v, sf0, sb0), (f1_v, b1_v, sf1, sb1))

    def issue(k, phase):
        fv, bv, sf, sb = bufs[phase]
        cb = k * VC
        pltpu.async_copy(tab.at[idxf_v.at[pl.ds(cb, VC)]], fv, sf)
        pltpu.async_copy(tab.at[idxb_v.at[pl.ds(cb, VC)]], bv, sb)

    issue(0, 0)
    issue(1, 1)

    def step(i, carry):
        for phase in range(2):
            k = i * 2 + phase
            fv, bv, sf, sb = bufs[phase]
            cb = k * VC
            pltpu.make_async_copy(tab.at[idxf_v.at[pl.ds(cb, VC)]], fv, sf).wait()
            pltpu.make_async_copy(tab.at[idxb_v.at[pl.ds(cb, VC)]], bv, sb).wait()
            _blend_chunk(o_v, fv, bv, wf_v, wb_v, cb, zeros)
            pltpu.sync_copy(o_v, out.at[pl.ds((base + cb) * D, VC * D)])

            @pl.when(k + 2 < NCH)
            def _():
                issue(k + 2, phase)

        return carry

    lax.fori_loop(0, NCH // 2, step, 0)


def _make_sample():
    return pl.kernel(
        _body,
        mesh=plsc.VectorSubcoreMesh(core_axis_name="c", subcore_axis_name="s"),
        out_type=jax.ShapeDtypeStruct((NVP * D,), jnp.float32),
        scratch_types=[
            pltpu.VMEM((CH,), jnp.int32),
            pltpu.VMEM((CH,), jnp.int32),
            pltpu.VMEM((CH,), jnp.int32),
            pltpu.VMEM((CH,), jnp.int32),
            pltpu.VMEM((CH + 16,), jnp.float32),
            pltpu.VMEM((CH + 16,), jnp.float32),
            pltpu.VMEM((VC, TD), jnp.float32),
            pltpu.VMEM((VC, TD), jnp.float32),
            pltpu.VMEM((VC, TD), jnp.float32),
            pltpu.VMEM((VC, TD), jnp.float32),
            pltpu.VMEM((VC * D,), jnp.float32),
            pltpu.SemaphoreType.DMA,
            pltpu.SemaphoreType.DMA,
            pltpu.SemaphoreType.DMA,
            pltpu.SemaphoreType.DMA,
        ],
    )


TT = 1024


def _tc_build(img_ref, out_ref):
    parts = [img_ref[:, c, :] for c in range(C)]
    parts.append(jnp.zeros((TD - B * C, TT), jnp.float32))
    blk = jnp.concatenate(parts, axis=0)
    out_ref[...] = jnp.transpose(blk, (1, 0))


def _build_table(img):
    return pl.pallas_call(
        _tc_build,
        grid=(HW // TT,),
        in_specs=[pl.BlockSpec((B, C, TT), lambda i: (0, 0, i))],
        out_specs=pl.BlockSpec((TT, TD), lambda i: (i, 0)),
        out_shape=jax.ShapeDtypeStruct((HW, TD), jnp.float32),
    )(img)


def kernel(offset_imgs, vt_idx_front, vt_idx_back, mask_front, mask_back):
    tab = _build_table(offset_imgs.reshape(B, C, HW))
    pad = NVP - NV
    idxf = jnp.pad(vt_idx_front, (0, pad), mode="wrap")
    idxb = jnp.pad(vt_idx_back, (0, pad), mode="wrap")
    mf = jnp.pad(mask_front, (0, pad))
    mb = jnp.pad(mask_back, (0, pad))
    o = _make_sample()(tab, idxf, idxb, mf, mb)
    return jnp.transpose(o.reshape(NVP, 3, B), (2, 0, 1))[:, :NV, :]


# speedup vs baseline: 1.4057x; 1.1342x over previous
import jax
import jax.numpy as jnp
from jax import lax
from jax.experimental import pallas as pl
from jax.experimental.pallas import tpu as pltpu
from jax.experimental.pallas import tpu_sc as plsc

B = 16
C = 6
HW = 512 * 512
NV = 100000
NC = 2
NS = 16
NW = NC * NS
CH = 3200
NVP = NW * CH
VC = 64
NCH = CH // VC
D = 48
TD = 128


def _blend_chunk(o_v, f_v, b_v, wf_v, wb_v, cb, zeros):
    def blend(r, cc):
        w1 = wf_v[pl.ds(cb + r, 16)].at[zeros].get(mode="promise_in_bounds")
        w2 = wb_v[pl.ds(cb + r, 16)].at[zeros].get(mode="promise_in_bounds")
        for j in range(3):
            s = pl.ds(r * D + j * 16, 16)
            o_v[s] = (f_v[r, pl.ds(j * 16, 16)] * w1
                      + b_v[r, pl.ds(D + j * 16, 16)] * w2)
        return cc

    lax.fori_loop(0, VC, blend, 0)


def _body(tab, idxf, idxb, mf, mb, out,
          idxf_v, idxb_v, mf_v, mb_v, wf_v, wb_v,
          f0_v, b0_v, f1_v, b1_v, o_v,
          sf0, sb0, sf1, sb1):
    cid = lax.axis_index("c")
    sid = lax.axis_index("s")
    wid = sid * NC + cid
    base = wid * CH

    pltpu.sync_copy(idxf.at[pl.ds(base, CH)], idxf_v)
    pltpu.sync_copy(idxb.at[pl.ds(base, CH)], idxb_v)
    pltpu.sync_copy(mf.at[pl.ds(base, CH)], mf_v)
    pltpu.sync_copy(mb.at[pl.ds(base, CH)], mb_v)

    def wloop(i, carry):
        s = pl.ds(i * 16, 16)
        a = lax.convert_element_type(mf_v[s], jnp.float32)
        bb = lax.convert_element_type(mb_v[s], jnp.float32)
        d = jnp.maximum(a + bb, 1.0)
        wf_v[s] = a / d
        wb_v[s] = bb / d
        return carry

    lax.fori_loop(0, CH // 16, wloop, 0)

    zeros = lax.shift_right_logical(lax.iota(jnp.int32, 16), 31)
    bufs = ((f0_v, b0_v, sf0, sb0), (f1_v, b1_v, sf1, sb1))

    def issue(k, phase):
        fv, bv, sf, sb = bufs[phase]
        cb = k * VC
        pltpu.async_copy(tab.at[idxf_v.at[pl.ds(cb, VC)]], fv, sf)
        pltpu.async_copy(tab.at[idxb_v.at[pl.ds(cb, VC)]], bv, sb)

    issue(0, 0)
    issue(1, 1)

    def step(i, carry):
        for phase in range(2):
            k = i * 2 + phase
            fv, bv, sf, sb = bufs[phase]
            cb = k * VC
            pltpu.make_async_copy(tab.at[idxf_v.at[pl.ds(cb, VC)]], fv, sf).wait()
            pltpu.make_async_copy(tab.at[idxb_v.at[pl.ds(cb, VC)]], bv, sb).wait()
            _blend_chunk(o_v, fv, bv, wf_v, wb_v, cb, zeros)
            pltpu.sync_copy(o_v, out.at[pl.ds((base + cb) * D, VC * D)])

            @pl.when(k + 2 < NCH)
            def _():
                issue(k + 2, phase)

        return carry

    lax.fori_loop(0, NCH // 2, step, 0)


def _make_sample():
    return pl.kernel(
        _body,
        mesh=plsc.VectorSubcoreMesh(core_axis_name="c", subcore_axis_name="s"),
        out_type=jax.ShapeDtypeStruct((NVP * D,), jnp.float32),
        scratch_types=[
            pltpu.VMEM((CH,), jnp.int32),
            pltpu.VMEM((CH,), jnp.int32),
            pltpu.VMEM((CH,), jnp.int32),
            pltpu.VMEM((CH,), jnp.int32),
            pltpu.VMEM((CH + 16,), jnp.float32),
            pltpu.VMEM((CH + 16,), jnp.float32),
            pltpu.VMEM((VC, TD), jnp.float32),
            pltpu.VMEM((VC, TD), jnp.float32),
            pltpu.VMEM((VC, TD), jnp.float32),
            pltpu.VMEM((VC, TD), jnp.float32),
            pltpu.VMEM((VC * D,), jnp.float32),
            pltpu.SemaphoreType.DMA,
            pltpu.SemaphoreType.DMA,
            pltpu.SemaphoreType.DMA,
            pltpu.SemaphoreType.DMA,
        ],
    )


TT = 4096


def _tc_build(img_ref, out_ref):
    parts = [img_ref[:, c, :] for c in range(C)]
    parts.append(jnp.zeros((TD - B * C, TT), jnp.float32))
    blk = jnp.concatenate(parts, axis=0)
    out_ref[...] = jnp.transpose(blk, (1, 0))


def _build_table(img):
    return pl.pallas_call(
        _tc_build,
        grid=(HW // TT,),
        in_specs=[pl.BlockSpec((B, C, TT), lambda i: (0, 0, i))],
        out_specs=pl.BlockSpec((TT, TD), lambda i: (i, 0)),
        out_shape=jax.ShapeDtypeStruct((HW, TD), jnp.float32),
    )(img)


def kernel(offset_imgs, vt_idx_front, vt_idx_back, mask_front, mask_back):
    tab = _build_table(offset_imgs.reshape(B, C, HW))
    pad = NVP - NV
    idxf = jnp.pad(vt_idx_front, (0, pad), mode="wrap")
    idxb = jnp.pad(vt_idx_back, (0, pad), mode="wrap")
    mf = jnp.pad(mask_front, (0, pad))
    mb = jnp.pad(mask_back, (0, pad))
    o = _make_sample()(tab, idxf, idxb, mf, mb)
    return jnp.transpose(o.reshape(NVP, 3, B), (2, 0, 1))[:, :NV, :]


# speedup vs baseline: 1.4439x; 1.0272x over previous
import jax
import jax.numpy as jnp
from jax import lax
from jax.experimental import pallas as pl
from jax.experimental.pallas import tpu as pltpu
from jax.experimental.pallas import tpu_sc as plsc

B = 16
C = 6
HW = 512 * 512
NV = 100000
NC = 2
NS = 16
NW = NC * NS
CH = 3200
NVP = NW * CH
VC = 64
NCH = CH // VC
D = 48
TD = 128


def _blend_chunk(o_v, f_v, b_v, wf_v, wb_v, cb, zeros):
    def blend(r, cc):
        w1 = wf_v[pl.ds(cb + r, 16)].at[zeros].get(mode="promise_in_bounds")
        w2 = wb_v[pl.ds(cb + r, 16)].at[zeros].get(mode="promise_in_bounds")
        for j in range(3):
            s = pl.ds(r * D + j * 16, 16)
            o_v[s] = (f_v[r, pl.ds(j * 16, 16)] * w1
                      + b_v[r, pl.ds(D + j * 16, 16)] * w2)
        return cc

    lax.fori_loop(0, VC, blend, 0)


def _body(tab, idxf, idxb, mf, mb, out,
          idxf_v, idxb_v, mf_v, mb_v, wf_v, wb_v,
          f0_v, b0_v, f1_v, b1_v, o_v,
          sf0, sb0, sf1, sb1):
    cid = lax.axis_index("c")
    sid = lax.axis_index("s")
    wid = sid * NC + cid
    base = wid * CH

    pltpu.sync_copy(idxf.at[pl.ds(base, CH)], idxf_v)
    pltpu.sync_copy(idxb.at[pl.ds(base, CH)], idxb_v)
    pltpu.sync_copy(mf.at[pl.ds(base, CH)], mf_v)
    pltpu.sync_copy(mb.at[pl.ds(base, CH)], mb_v)

    def wloop(i, carry):
        s = pl.ds(i * 16, 16)
        a = lax.convert_element_type(mf_v[s], jnp.float32)
        bb = lax.convert_element_type(mb_v[s], jnp.float32)
        d = jnp.maximum(a + bb, 1.0)
        wf_v[s] = a / d
        wb_v[s] = bb / d
        return carry

    lax.fori_loop(0, CH // 16, wloop, 0)

    zeros = lax.shift_right_logical(lax.iota(jnp.int32, 16), 31)
    bufs = ((f0_v, b0_v, sf0, sb0), (f1_v, b1_v, sf1, sb1))

    def issue(k, phase):
        fv, bv, sf, sb = bufs[phase]
        cb = k * VC
        pltpu.async_copy(tab.at[idxf_v.at[pl.ds(cb, VC)]], fv, sf)
        pltpu.async_copy(tab.at[idxb_v.at[pl.ds(cb, VC)]], bv, sb)

    issue(0, 0)
    issue(1, 1)

    def step(i, carry):
        for phase in range(2):
            k = i * 2 + phase
            fv, bv, sf, sb = bufs[phase]
            cb = k * VC
            pltpu.make_async_copy(tab.at[idxf_v.at[pl.ds(cb, VC)]], fv, sf).wait()
            pltpu.make_async_copy(tab.at[idxb_v.at[pl.ds(cb, VC)]], bv, sb).wait()
            _blend_chunk(o_v, fv, bv, wf_v, wb_v, cb, zeros)
            pltpu.sync_copy(o_v, out.at[pl.ds((base + cb) * D, VC * D)])

            @pl.when(k + 2 < NCH)
            def _():
                issue(k + 2, phase)

        return carry

    lax.fori_loop(0, NCH // 2, step, 0)


def _make_sample():
    return pl.kernel(
        _body,
        mesh=plsc.VectorSubcoreMesh(core_axis_name="c", subcore_axis_name="s"),
        out_type=jax.ShapeDtypeStruct((NVP * D,), jnp.float32),
        scratch_types=[
            pltpu.VMEM((CH,), jnp.int32),
            pltpu.VMEM((CH,), jnp.int32),
            pltpu.VMEM((CH,), jnp.int32),
            pltpu.VMEM((CH,), jnp.int32),
            pltpu.VMEM((CH + 16,), jnp.float32),
            pltpu.VMEM((CH + 16,), jnp.float32),
            pltpu.VMEM((VC, TD), jnp.float32),
            pltpu.VMEM((VC, TD), jnp.float32),
            pltpu.VMEM((VC, TD), jnp.float32),
            pltpu.VMEM((VC, TD), jnp.float32),
            pltpu.VMEM((VC * D,), jnp.float32),
            pltpu.SemaphoreType.DMA,
            pltpu.SemaphoreType.DMA,
            pltpu.SemaphoreType.DMA,
            pltpu.SemaphoreType.DMA,
        ],
    )


TT = 16384


def _tc_build(img_ref, out_ref):
    parts = [img_ref[:, c, :] for c in range(C)]
    parts.append(jnp.zeros((TD - B * C, TT), jnp.float32))
    blk = jnp.concatenate(parts, axis=0)
    out_ref[...] = jnp.transpose(blk, (1, 0))


def _build_table(img):
    return pl.pallas_call(
        _tc_build,
        grid=(HW // TT,),
        in_specs=[pl.BlockSpec((B, C, TT), lambda i: (0, 0, i))],
        out_specs=pl.BlockSpec((TT, TD), lambda i: (i, 0)),
        out_shape=jax.ShapeDtypeStruct((HW, TD), jnp.float32),
    )(img)


def kernel(offset_imgs, vt_idx_front, vt_idx_back, mask_front, mask_back):
    tab = _build_table(offset_imgs.reshape(B, C, HW))
    pad = NVP - NV
    idxf = jnp.pad(vt_idx_front, (0, pad), mode="wrap")
    idxb = jnp.pad(vt_idx_back, (0, pad), mode="wrap")
    mf = jnp.pad(mask_front, (0, pad))
    mb = jnp.pad(mask_back, (0, pad))
    o = _make_sample()(tab, idxf, idxb, mf, mb)
    return jnp.transpose(o.reshape(NVP, 3, B), (2, 0, 1))[:, :NV, :]
